# baseline (speedup 1.0000x reference)
import functools

import jax
import jax.numpy as jnp
from jax import lax
from jax.experimental import pallas as pl
from jax.experimental.pallas import tpu as pltpu
from jax.experimental.pallas import tpu_sc as plsc

_N = 20000
_C = 81
_D = 5
_J = 405
_NBANDS = 51
_NC = 2
_NS = 16
_NW = _NC * _NS
_L = 16
_CW = 128
_NCHUNK = 157
_NPAD = _NCHUNK * _CW
_PASSES = 5


def _body(tabt_hbm, labels_hbm, out_hbm, bands0_v, bands1_v, lab_v, rows_v,
          sem0, sem1):
    wid = lax.axis_index("s") * _NC + lax.axis_index("c")
    lane = lax.iota(jnp.int32, _L)

    def band_slices(p, buf):
        chunk = p * _NW + wid
        i0 = pl.multiple_of(chunk * _CW, _CW)
        for b in range(_NBANDS):
            span = min(8, _J - 8 * b)
            yield (
                tabt_hbm.at[pl.ds(b * 8, span), pl.ds(i0, _CW)],
                buf.at[b, pl.ds(0, span)],
            )

    def start(p, buf, sem):
        chunk = p * _NW + wid

        @pl.when(chunk < _NCHUNK)
        def _():
            for src, dst in band_slices(p, buf):
                pltpu.async_copy(src, dst, sem)

    def drain(p, buf, sem):
        chunk = p * _NW + wid

        @pl.when(chunk < _NCHUNK)
        def _():
            for src, dst in band_slices(p, buf):
                pltpu.make_async_copy(src, dst, sem).wait()

    def extract(p, buf):
        chunk = p * _NW + wid

        @pl.when(chunk < _NCHUNK)
        def _():
            i0 = pl.multiple_of(chunk * _CW, _CW)
            for g in range(_CW // _L):
                ilocal = g * _L + lane
                lab = lab_v[pl.ds(i0 + g * _L, _L)]
                j0 = jnp.clip(lab, 0, _C - 1) * _D
                band = lax.shift_right_logical(j0, 3)
                woff = jnp.bitwise_and(j0, 7)
                for j in range(_D):
                    p_ = woff + j
                    bandsel = band + lax.shift_right_logical(p_, 3)
                    wordsel = jnp.bitwise_and(p_, 7)
                    val = plsc.load_gather(buf, [bandsel, wordsel, ilocal])
                    plsc.store_scatter(
                        rows_v, [jnp.full((_L,), j, jnp.int32), ilocal], val
                    )
            pltpu.sync_copy(rows_v, out_hbm.at[:, pl.ds(i0, _CW)])

    start(0, bands0_v, sem0)
    pltpu.sync_copy(labels_hbm, lab_v)

    def pair_body(q, _):
        pa = 2 * q
        pb = 2 * q + 1
        drain(pa, bands0_v, sem0)
        start(pb, bands1_v, sem1)
        extract(pa, bands0_v)
        drain(pb, bands1_v, sem1)
        start(pa + 2, bands0_v, sem0)
        extract(pb, bands1_v)
        return 0

    lax.fori_loop(0, (_PASSES + 1) // 2, pair_body, 0)


@jax.jit
def _run(tabt, labels32):
    mesh = plsc.VectorSubcoreMesh(core_axis_name="c", subcore_axis_name="s")
    f = functools.partial(
        pl.kernel,
        out_type=jax.ShapeDtypeStruct((_D, _NPAD), jnp.float32),
        mesh=mesh,
        scratch_types=[
            pltpu.VMEM((_NBANDS, 8, _CW), jnp.float32),
            pltpu.VMEM((_NBANDS, 8, _CW), jnp.float32),
            pltpu.VMEM((_N,), jnp.int32),
            pltpu.VMEM((_D, _CW), jnp.float32),
            pltpu.SemaphoreType.DMA,
            pltpu.SemaphoreType.DMA,
        ],
        compiler_params=pltpu.CompilerParams(
            use_tc_tiling_on_sc=True,
            disable_bounds_checks=True,
            needs_layout_passes=False,
        ),
    )(_body)
    return f(tabt, labels32)


def kernel(pose_pred, labels):
    out_t = _run(pose_pred.T, labels.astype(jnp.int32))
    return out_t.T[:_N]
